# baseline (speedup 1.0000x reference)
import jax
import jax.numpy as jnp
from jax.experimental import pallas as pl

NSEG = 16
FOLD = 2
BLK = 8192
BLK2 = BLK // FOLD
DF = 61 * FOLD
DP = 3 * FOLD
DO = 64 * FOLD


def _pool_kernel(x_ref, pos_ref, w1_ref, w2_ref, b_ref, batch_ref, out_ref):
    i = pl.program_id(0)
    nblk = pl.num_programs(0)

    @pl.when(i == 0)
    def _init():
        out_ref[...] = jnp.full((NSEG, 64), -jnp.inf, dtype=jnp.float32)

    z = jnp.dot(x_ref[...], w1_ref[...], preferred_element_type=jnp.float32)
    z = z + jnp.dot(pos_ref[...], w2_ref[...], preferred_element_type=jnp.float32)

    bb = batch_ref[...]
    lo = batch_ref[0, 0]
    hi = batch_ref[BLK2 - 1, 1]

    @pl.when(lo == hi)
    def _pure():
        v = jnp.max(z, axis=0, keepdims=True)
        v = jnp.maximum(v[:, :64], v[:, 64:])
        cur = out_ref[pl.ds(lo, 1), :]
        out_ref[pl.ds(lo, 1), :] = jnp.maximum(cur, v)

    @pl.when(lo != hi)
    def _mixed():
        lane = jax.lax.broadcasted_iota(jnp.int32, (BLK2, DO), 1)
        bl = jnp.where(lane < 64, bb[:, 0:1], bb[:, 1:2])
        for s in range(NSEG):
            @pl.when(jnp.logical_and(lo <= s, s <= hi))
            def _acc(s=s):
                v = jnp.max(jnp.where(bl == s, z, -jnp.inf), axis=0,
                            keepdims=True)
                v = jnp.maximum(v[:, :64], v[:, 64:])
                out_ref[s:s + 1, :] = jnp.maximum(out_ref[s:s + 1, :], v)

    @pl.when(i == nblk - 1)
    def _finish():
        acc = out_ref[...]
        res = jnp.maximum(acc + b_ref[...], 0.0)
        out_ref[...] = jnp.where(acc == -jnp.inf, acc, res)


def kernel(x, pos, W, b, batch):
    n = x.shape[0]
    n2 = n // FOLD
    nblk = n2 // BLK2

    w1 = W[:, :61].T
    w2 = W[:, 61:].T
    z61 = jnp.zeros((61, 64), jnp.float32)
    z3 = jnp.zeros((3, 64), jnp.float32)
    w1b = jnp.concatenate(
        [jnp.concatenate([w1, z61], axis=1),
         jnp.concatenate([z61, w1], axis=1)], axis=0)
    w2b = jnp.concatenate(
        [jnp.concatenate([w2, z3], axis=1),
         jnp.concatenate([z3, w2], axis=1)], axis=0)
    b2 = b.reshape(1, 64)

    x2 = x.reshape(n2, DF)
    pos2 = pos.reshape(n2, DP)
    batch2 = batch.astype(jnp.int32).reshape(n2, FOLD)

    return pl.pallas_call(
        _pool_kernel,
        grid=(nblk,),
        in_specs=[
            pl.BlockSpec((BLK2, DF), lambda i: (i, 0)),
            pl.BlockSpec((BLK2, DP), lambda i: (i, 0)),
            pl.BlockSpec((DF, DO), lambda i: (0, 0)),
            pl.BlockSpec((DP, DO), lambda i: (0, 0)),
            pl.BlockSpec((1, 64), lambda i: (0, 0)),
            pl.BlockSpec((BLK2, FOLD), lambda i: (i, 0)),
        ],
        out_specs=pl.BlockSpec((NSEG, 64), lambda i: (0, 0)),
        out_shape=jax.ShapeDtypeStruct((NSEG, 64), jnp.float32),
    )(x2, pos2, w1b, w2b, b2, batch2)


# speedup vs baseline: 1.7651x; 1.7651x over previous
import jax
import jax.numpy as jnp
from jax import lax
from jax.experimental import pallas as pl

NSEG = 16
BLK = 8192
BPR = BLK // 128


def _pool_kernel(x_ref, pos_ref, w1_ref, w2_ref, b_ref, batch_ref, out_ref):
    i = pl.program_id(0)
    nblk = pl.num_programs(0)

    @pl.when(i == 0)
    def _init():
        out_ref[...] = jnp.full((NSEG, 64), -jnp.inf, dtype=jnp.float32)

    z = jnp.dot(x_ref[...], w1_ref[...], preferred_element_type=jnp.float32)
    z = z + jnp.dot(pos_ref[...], w2_ref[...], preferred_element_type=jnp.float32)
    zr = z.reshape(BLK // 8, 8, 64)

    bb = batch_ref[...]
    lo = batch_ref[0, 0]
    hi = batch_ref[BPR - 1, 127]

    @pl.when(lo == hi)
    def _pure():
        v8 = jnp.max(zr, axis=0)
        v = jnp.max(v8, axis=0, keepdims=True)
        cur = out_ref[pl.ds(lo, 1), :]
        out_ref[pl.ds(lo, 1), :] = jnp.maximum(cur, v)

    @pl.when(lo != hi)
    def _mixed():
        riota = (lax.broadcasted_iota(jnp.int32, (BLK // 8, 8, 1), 0) * 8
                 + lax.broadcasted_iota(jnp.int32, (BLK // 8, 8, 1), 1))
        for s in range(NSEG):
            @pl.when(jnp.logical_and(lo <= s, s <= hi))
            def _acc(s=s):
                start = jnp.sum((bb < s).astype(jnp.int32))
                end = jnp.sum((bb <= s).astype(jnp.int32))
                m = jnp.logical_and(riota >= start, riota < end)
                v8 = jnp.max(jnp.where(m, zr, -jnp.inf), axis=0)
                v = jnp.max(v8, axis=0, keepdims=True)
                out_ref[s:s + 1, :] = jnp.maximum(out_ref[s:s + 1, :], v)

    @pl.when(i == nblk - 1)
    def _finish():
        acc = out_ref[...]
        res = jnp.maximum(acc + b_ref[...], 0.0)
        out_ref[...] = jnp.where(acc == -jnp.inf, acc, res)


def kernel(x, pos, W, b, batch):
    n = x.shape[0]
    nblk = n // BLK

    w1 = W[:, :61].T
    w2 = W[:, 61:].T
    b2 = b.reshape(1, 64)
    batchv = batch.astype(jnp.int32).reshape(n // 128, 128)

    return pl.pallas_call(
        _pool_kernel,
        grid=(nblk,),
        in_specs=[
            pl.BlockSpec((BLK, 61), lambda i: (i, 0)),
            pl.BlockSpec((BLK, 3), lambda i: (i, 0)),
            pl.BlockSpec((61, 64), lambda i: (0, 0)),
            pl.BlockSpec((3, 64), lambda i: (0, 0)),
            pl.BlockSpec((1, 64), lambda i: (0, 0)),
            pl.BlockSpec((BPR, 128), lambda i: (i, 0)),
        ],
        out_specs=pl.BlockSpec((NSEG, 64), lambda i: (0, 0)),
        out_shape=jax.ShapeDtypeStruct((NSEG, 64), jnp.float32),
    )(x, pos, w1, w2, b2, batchv)


# speedup vs baseline: 1.7667x; 1.0009x over previous
import jax
import jax.numpy as jnp
from jax import lax
from jax.experimental import pallas as pl

NSEG = 16
BLK = 8192
BPR = BLK // 128


def _pool_kernel(x_ref, pos_ref, w1_ref, w2_ref, b_ref, batch_ref, out_ref):
    i = pl.program_id(0)
    nblk = pl.num_programs(0)

    @pl.when(i == 0)
    def _init():
        out_ref[...] = jnp.full((NSEG, 64), -jnp.inf, dtype=jnp.float32)

    z = jnp.dot(x_ref[...], w1_ref[...], preferred_element_type=jnp.float32)
    z = z + jnp.dot(pos_ref[...], w2_ref[...], preferred_element_type=jnp.float32)

    bb = batch_ref[...]
    lo = batch_ref[0, 0]
    hi = batch_ref[BPR - 1, 127]

    def _treemax(t):
        r = t.shape[0]
        while r > 8:
            r //= 2
            t = jnp.maximum(t[:r], t[r:])
        return jnp.max(t, axis=0, keepdims=True)

    @pl.when(lo == hi)
    def _pure():
        v = _treemax(z)
        cur = out_ref[pl.ds(lo, 1), :]
        out_ref[pl.ds(lo, 1), :] = jnp.maximum(cur, v)

    @pl.when(lo != hi)
    def _mixed():
        riota = lax.broadcasted_iota(jnp.int32, (BLK, 1), 0)
        for s in range(NSEG):
            @pl.when(jnp.logical_and(lo <= s, s <= hi))
            def _acc(s=s):
                start = jnp.sum((bb < s).astype(jnp.int32))
                end = jnp.sum((bb <= s).astype(jnp.int32))
                m = jnp.logical_and(riota >= start, riota < end)
                v = _treemax(jnp.where(m, z, -jnp.inf))
                out_ref[s:s + 1, :] = jnp.maximum(out_ref[s:s + 1, :], v)

    @pl.when(i == nblk - 1)
    def _finish():
        acc = out_ref[...]
        res = jnp.maximum(acc + b_ref[...], 0.0)
        out_ref[...] = jnp.where(acc == -jnp.inf, acc, res)


def kernel(x, pos, W, b, batch):
    n = x.shape[0]
    nblk = n // BLK

    w1 = W[:, :61].T
    w2 = W[:, 61:].T
    b2 = b.reshape(1, 64)
    batchv = batch.astype(jnp.int32).reshape(n // 128, 128)

    return pl.pallas_call(
        _pool_kernel,
        grid=(nblk,),
        in_specs=[
            pl.BlockSpec((BLK, 61), lambda i: (i, 0)),
            pl.BlockSpec((BLK, 3), lambda i: (i, 0)),
            pl.BlockSpec((61, 64), lambda i: (0, 0)),
            pl.BlockSpec((3, 64), lambda i: (0, 0)),
            pl.BlockSpec((1, 64), lambda i: (0, 0)),
            pl.BlockSpec((BPR, 128), lambda i: (i, 0)),
        ],
        out_specs=pl.BlockSpec((NSEG, 64), lambda i: (0, 0)),
        out_shape=jax.ShapeDtypeStruct((NSEG, 64), jnp.float32),
    )(x, pos, w1, w2, b2, batchv)
